# baseline (speedup 1.0000x reference)
import functools

import numpy as np
import jax
import jax.numpy as jnp
from jax import lax
from jax.experimental import pallas as pl
from jax.experimental.pallas import tpu as pltpu
from jax.experimental.pallas import tpu_sc as plsc

MAX_LEN = 200
MODEL_DIM = 64
BATCH = 1024

NC = 2
NS = 16
NW = NC * NS

ROWS = BATCH * MAX_LEN
ROWS_PER_W = ROWS // NW
SEQ_PER_W = ROWS_PER_W // MAX_LEN
IDX_MINOR = 100
IDX_ROWS_PER_W = ROWS_PER_W // IDX_MINOR
VECS_PER_ROW = MODEL_DIM // 16


def _make_pe_np():
    pos = np.arange(MAX_LEN)[:, None]
    pe = pos / np.power(10000, 2.0 * np.arange(MODEL_DIM)[None, :] / MODEL_DIM)
    pe[:, 0::2] = np.sin(pe[:, 0::2])
    pe[:, 1::2] = np.cos(pe[:, 1::2])
    return np.asarray(pe, dtype=np.float32)


_PE = _make_pe_np()

_mesh = plsc.VectorSubcoreMesh(
    core_axis_name="c", subcore_axis_name="s", num_cores=NC, num_subcores=NS
)


@functools.partial(
    pl.kernel,
    out_type=jax.ShapeDtypeStruct((ROWS, MODEL_DIM), jnp.float32),
    mesh=_mesh,
    scratch_types=[
        pltpu.VMEM((IDX_ROWS_PER_W, IDX_MINOR), jnp.int32),
        pltpu.VMEM((MAX_LEN, MODEL_DIM), jnp.float32),
        pltpu.VMEM((MAX_LEN, MODEL_DIM), jnp.float32),
        pltpu.VMEM((MAX_LEN, MODEL_DIM), jnp.float32),
        pltpu.VMEM((MAX_LEN, MODEL_DIM), jnp.float32),
        pltpu.VMEM((MAX_LEN, MODEL_DIM), jnp.float32),
        pltpu.SemaphoreType.DMA,
        pltpu.SemaphoreType.DMA,
        pltpu.SemaphoreType.DMA,
        pltpu.SemaphoreType.DMA,
    ],
    compiler_params=pltpu.CompilerParams(use_tc_tiling_on_sc=False),
)
def _sc_embed(x_ref, pe_ref, table_ref, out_ref,
              idx_v, pe_v, gb0, gb1, sb0, sb1,
              gsem0, gsem1, ssem0, ssem1):
    cid = lax.axis_index("c")
    sid = lax.axis_index("s")
    wid = sid * NC + cid
    row_base = wid * ROWS_PER_W

    gb = (gb0, gb1)
    sb = (sb0, sb1)
    gsem = (gsem0, gsem1)
    ssem = (ssem0, ssem1)

    pltpu.sync_copy(x_ref.at[pl.ds(wid * IDX_ROWS_PER_W, IDX_ROWS_PER_W)], idx_v)
    pltpu.sync_copy(pe_ref, pe_v)

    def gather(k, seq):
        pltpu.async_copy(table_ref.at[idx_v.at[2 * seq]],
                         gb[k].at[pl.ds(0, IDX_MINOR)], gsem[k])
        pltpu.async_copy(table_ref.at[idx_v.at[2 * seq + 1]],
                         gb[k].at[pl.ds(IDX_MINOR, IDX_MINOR)], gsem[k])

    def wait_gather(k, seq):
        pltpu.make_async_copy(table_ref.at[idx_v.at[2 * seq]],
                              gb[k].at[pl.ds(0, IDX_MINOR)], gsem[k]).wait()
        pltpu.make_async_copy(table_ref.at[idx_v.at[2 * seq + 1]],
                              gb[k].at[pl.ds(IDX_MINOR, IDX_MINOR)], gsem[k]).wait()

    def store(k, seq):
        pltpu.async_copy(sb[k], out_ref.at[pl.ds(row_base + seq * MAX_LEN, MAX_LEN)],
                         ssem[k])

    def wait_store(k, seq):
        pltpu.make_async_copy(sb[k],
                              out_ref.at[pl.ds(row_base + seq * MAX_LEN, MAX_LEN)],
                              ssem[k]).wait()

    def add_pe(k):
        def row(r, carry):
            for c in range(VECS_PER_ROW):
                sl = pl.ds(c * 16, 16)
                sb[k][r, sl] = gb[k][r, sl] + pe_v[r, sl]
            return carry
        lax.fori_loop(0, MAX_LEN, row, 0)

    gather(0, 0)
    gather(1, 1)

    for k in range(2):
        wait_gather(k, k)
        add_pe(k)
        store(k, k)
        gather(k, k + 2)

    def steady(i, carry):
        for k in range(2):
            seq = 2 * i + k
            wait_gather(k, seq)
            wait_store(k, seq - 2)
            add_pe(k)
            store(k, seq)
            gather(k, seq + 2)
        return carry
    lax.fori_loop(1, SEQ_PER_W // 2 - 1, steady, 0)

    for k in range(2):
        seq = SEQ_PER_W - 2 + k
        wait_gather(k, seq)
        wait_store(k, seq - 2)
        add_pe(k)
        store(k, seq)

    for k in range(2):
        wait_store(k, SEQ_PER_W - 2 + k)


def kernel(x, table):
    x2d = x.reshape(ROWS // IDX_MINOR, IDX_MINOR)
    pe = jnp.asarray(_PE)
    out = _sc_embed(x2d, pe, table)
    return out.reshape(BATCH, MAX_LEN, MODEL_DIM)


# speedup vs baseline: 1.0004x; 1.0004x over previous
import functools

import numpy as np
import jax
import jax.numpy as jnp
from jax import lax
from jax.experimental import pallas as pl
from jax.experimental.pallas import tpu as pltpu
from jax.experimental.pallas import tpu_sc as plsc

MAX_LEN = 200
MODEL_DIM = 64
BATCH = 1024

NC = 2
NS = 16
NW = NC * NS

ROWS = BATCH * MAX_LEN
ROWS_PER_W = ROWS // NW
IDX_MINOR = 100
IDX_ROWS_PER_W = ROWS_PER_W // IDX_MINOR
VECS_PER_ROW = MODEL_DIM // 16

CHUNK = 400
NCHUNK = ROWS_PER_W // CHUNK
IDX_PER_CHUNK = CHUNK // IDX_MINOR
NBUF = 4


def _make_pe_np():
    pos = np.arange(MAX_LEN)[:, None]
    pe = pos / np.power(10000, 2.0 * np.arange(MODEL_DIM)[None, :] / MODEL_DIM)
    pe[:, 0::2] = np.sin(pe[:, 0::2])
    pe[:, 1::2] = np.cos(pe[:, 1::2])
    return np.asarray(pe, dtype=np.float32)


_PE = _make_pe_np()

_mesh = plsc.VectorSubcoreMesh(
    core_axis_name="c", subcore_axis_name="s", num_cores=NC, num_subcores=NS
)


@functools.partial(
    pl.kernel,
    out_type=jax.ShapeDtypeStruct((ROWS, MODEL_DIM), jnp.float32),
    mesh=_mesh,
    scratch_types=[
        pltpu.VMEM((IDX_ROWS_PER_W, IDX_MINOR), jnp.int32),
        pltpu.VMEM((MAX_LEN, MODEL_DIM), jnp.float32),
    ]
    + [pltpu.VMEM((CHUNK, MODEL_DIM), jnp.float32) for _ in range(NBUF)]
    + [pltpu.SemaphoreType.DMA for _ in range(2 * NBUF)],
    compiler_params=pltpu.CompilerParams(use_tc_tiling_on_sc=False),
)
def _sc_embed(x_ref, pe_ref, table_ref, out_ref,
              idx_v, pe_v, b0, b1, b2, b3,
              g0, g1, g2, g3, s0, s1, s2, s3):
    cid = lax.axis_index("c")
    sid = lax.axis_index("s")
    wid = sid * NC + cid
    row_base = wid * ROWS_PER_W

    buf = (b0, b1, b2, b3)
    gsem = (g0, g1, g2, g3)
    ssem = (s0, s1, s2, s3)

    pltpu.sync_copy(x_ref.at[pl.ds(wid * IDX_ROWS_PER_W, IDX_ROWS_PER_W)], idx_v)
    pltpu.sync_copy(pe_ref, pe_v)

    def gather(c):
        k = c % NBUF
        for j in range(IDX_PER_CHUNK):
            pltpu.async_copy(table_ref.at[idx_v.at[IDX_PER_CHUNK * c + j]],
                             buf[k].at[pl.ds(j * IDX_MINOR, IDX_MINOR)],
                             gsem[k])

    def wait_gather(c):
        k = c % NBUF
        for j in range(IDX_PER_CHUNK):
            pltpu.make_async_copy(table_ref.at[idx_v.at[IDX_PER_CHUNK * c + j]],
                                  buf[k].at[pl.ds(j * IDX_MINOR, IDX_MINOR)],
                                  gsem[k]).wait()

    def store(c):
        k = c % NBUF
        pltpu.async_copy(buf[k], out_ref.at[pl.ds(row_base + c * CHUNK, CHUNK)],
                         ssem[k])

    def wait_store(c):
        k = c % NBUF
        pltpu.make_async_copy(buf[k],
                              out_ref.at[pl.ds(row_base + c * CHUNK, CHUNK)],
                              ssem[k]).wait()

    def add_pe(c):
        k = c % NBUF

        @plsc.parallel_loop(0, MAX_LEN, unroll=4)
        def _row(r):
            for h in range(CHUNK // MAX_LEN):
                for v in range(VECS_PER_ROW):
                    sl = pl.ds(v * 16, 16)
                    plsc.addupdate(buf[k].at[h * MAX_LEN + r, sl], pe_v[r, sl])

    for c in range(min(NBUF - 1, NCHUNK)):
        gather(c)
    for c in range(NCHUNK):
        wait_gather(c)
        add_pe(c)
        store(c)
        nxt = c + NBUF - 1
        if nxt < NCHUNK:
            if c >= 1:
                wait_store(c - 1)
            gather(nxt)
    for c in range(max(NCHUNK - NBUF, 0), NCHUNK):
        wait_store(c)


def kernel(x, table):
    x2d = x.reshape(ROWS // IDX_MINOR, IDX_MINOR)
    pe = jnp.asarray(_PE)
    out = _sc_embed(x2d, pe, table)
    return out.reshape(BATCH, MAX_LEN, MODEL_DIM)


# speedup vs baseline: 1.0024x; 1.0021x over previous
import functools

import numpy as np
import jax
import jax.numpy as jnp
from jax import lax
from jax.experimental import pallas as pl
from jax.experimental.pallas import tpu as pltpu
from jax.experimental.pallas import tpu_sc as plsc

MAX_LEN = 200
MODEL_DIM = 64
BATCH = 1024

NC = 2
NS = 16
NW = NC * NS

SEQ_PER_W = BATCH // NW
IDX_MINOR = 100
VECS_PER_ROW = MODEL_DIM // 16

SEQ_PER_CHUNK = 2
NCHUNK = SEQ_PER_W // SEQ_PER_CHUNK
NBUF = 4


def _make_pe_np():
    pos = np.arange(MAX_LEN)[:, None]
    pe = pos / np.power(10000, 2.0 * np.arange(MODEL_DIM)[None, :] / MODEL_DIM)
    pe[:, 0::2] = np.sin(pe[:, 0::2])
    pe[:, 1::2] = np.cos(pe[:, 1::2])
    return np.asarray(pe, dtype=np.float32)


_PE = _make_pe_np()

_mesh = plsc.VectorSubcoreMesh(
    core_axis_name="c", subcore_axis_name="s", num_cores=NC, num_subcores=NS
)


@functools.partial(
    pl.kernel,
    out_type=jax.ShapeDtypeStruct((BATCH, MAX_LEN, MODEL_DIM), jnp.float32),
    mesh=_mesh,
    scratch_types=[
        pltpu.VMEM((SEQ_PER_W * MAX_LEN // IDX_MINOR, IDX_MINOR), jnp.int32),
        pltpu.VMEM((MAX_LEN, MODEL_DIM), jnp.float32),
    ]
    + [pltpu.VMEM((SEQ_PER_CHUNK * MAX_LEN, MODEL_DIM), jnp.float32)
       for _ in range(NBUF)]
    + [pltpu.SemaphoreType.DMA for _ in range(2 * NBUF)],
    compiler_params=pltpu.CompilerParams(use_tc_tiling_on_sc=False),
)
def _sc_embed(x_ref, pe_ref, table_ref, out_ref,
              idx_v, pe_v, b0, b1, b2, b3,
              g0, g1, g2, g3, s0, s1, s2, s3):
    cid = lax.axis_index("c")
    sid = lax.axis_index("s")
    wid = sid * NC + cid
    seq_base = wid * SEQ_PER_W

    buf = (b0, b1, b2, b3)
    gsem = (g0, g1, g2, g3)
    ssem = (s0, s1, s2, s3)

    idx_rows_per_w = SEQ_PER_W * MAX_LEN // IDX_MINOR
    idx_per_chunk = SEQ_PER_CHUNK * MAX_LEN // IDX_MINOR

    pltpu.sync_copy(x_ref.at[pl.ds(wid * idx_rows_per_w, idx_rows_per_w)],
                    idx_v)
    pltpu.sync_copy(pe_ref, pe_v)

    def _parts(c):
        k = c % NBUF
        for p in range(idx_per_chunk):
            yield (buf[k].at[pl.ds(p * IDX_MINOR, IDX_MINOR)],
                   idx_v.at[idx_per_chunk * c + p])

    def gather(c):
        k = c % NBUF
        for dst, idx in _parts(c):
            pltpu.async_copy(table_ref.at[idx], dst, gsem[k])

    def wait_gather(c):
        k = c % NBUF
        for dst, idx in _parts(c):
            pltpu.make_async_copy(table_ref.at[idx], dst, gsem[k]).wait()

    def store(c):
        k = c % NBUF
        for h in range(SEQ_PER_CHUNK):
            pltpu.async_copy(buf[k].at[pl.ds(h * MAX_LEN, MAX_LEN)],
                             out_ref.at[seq_base + c * SEQ_PER_CHUNK + h],
                             ssem[k])

    def wait_store(c):
        k = c % NBUF
        for h in range(SEQ_PER_CHUNK):
            pltpu.make_async_copy(buf[k].at[pl.ds(h * MAX_LEN, MAX_LEN)],
                                  out_ref.at[seq_base + c * SEQ_PER_CHUNK + h],
                                  ssem[k]).wait()

    def add_pe(c):
        k = c % NBUF

        @plsc.parallel_loop(0, MAX_LEN, unroll=4)
        def _row(r):
            for h in range(SEQ_PER_CHUNK):
                for v in range(VECS_PER_ROW):
                    sl = pl.ds(v * 16, 16)
                    plsc.addupdate(buf[k].at[h * MAX_LEN + r, sl], pe_v[r, sl])

    for c in range(min(NBUF - 1, NCHUNK)):
        gather(c)
    for c in range(NCHUNK):
        wait_gather(c)
        add_pe(c)
        store(c)
        nxt = c + NBUF - 1
        if nxt < NCHUNK:
            if c >= 1:
                wait_store(c - 1)
            gather(nxt)
    for c in range(max(NCHUNK - NBUF, 0), NCHUNK):
        wait_store(c)


def kernel(x, table):
    pe = jnp.asarray(_PE)
    x2d = x.reshape(BATCH * MAX_LEN // IDX_MINOR, IDX_MINOR)
    return _sc_embed(x2d, pe, table)
